# baseline (speedup 1.0000x reference)
import functools

import numpy as np
import jax
import jax.numpy as jnp
from jax.experimental import pallas as pl
from jax.experimental.pallas import tpu as pltpu

_S = 4096
_WIN = 16
_H = 16
_D = 64
_DIM = 1024
_BQ = 128
_BK = 128
_NEG = -1e30


def _hilbert_coords(seq_len):
    n = 1
    while n * n < seq_len:
        n *= 2
    t = np.arange(seq_len, dtype=np.int64)
    x = np.zeros(seq_len, dtype=np.int64)
    y = np.zeros(seq_len, dtype=np.int64)
    s = 1
    while s < n:
        rx = 1 & (t // 2)
        ry = 1 & (t ^ rx)
        swap = ry == 0
        flip = swap & (rx == 1)
        xf = np.where(flip, s - 1 - x, x)
        yf = np.where(flip, s - 1 - y, y)
        xn = np.where(swap, yf, xf)
        yn = np.where(swap, xf, yf)
        x = xn + s * rx
        y = yn + s * ry
        t = t // 4
        s *= 2
    return np.stack([x, y], axis=-1).astype(np.float32)


def _neighbor_indices(seq_len, window):
    coords = _hilbert_coords(seq_len)
    diff = coords[:, None, :] - coords[None, :, :]
    dist = np.sqrt((diff ** 2).sum(-1))
    order = np.argsort(dist, axis=-1, kind="stable")
    return order[:, : min(window, seq_len)]


def _build_schedule():
    ni = _neighbor_indices(_S, _WIN)
    qb = np.repeat(np.arange(_S) // _BQ, _WIN)
    kb = (ni // _BK).ravel()
    pairs = sorted(set(zip(qb.tolist(), kb.tolist())))
    P = len(pairs)
    tab = np.zeros((P, 4), dtype=np.int32)
    mask = np.full((P, _BQ, _BK), _NEG, dtype=np.float32)
    for p, (qi, ki) in enumerate(pairs):
        tab[p, 0] = qi
        tab[p, 1] = ki
        tab[p, 2] = int(p == 0 or pairs[p - 1][0] != qi)
        tab[p, 3] = int(p == P - 1 or pairs[p + 1][0] != qi)
        rows = ni[qi * _BQ:(qi + 1) * _BQ]
        r, w = np.nonzero((rows // _BK) == ki)
        mask[p, r, rows[r, w] - ki * _BK] = 0.0
    return tab, mask


_TAB_NP, _MASK_NP = _build_schedule()
_NUM_PAIRS = _TAB_NP.shape[0]


def _proj_kernel(x_ref, w_ref, q_ref, k_ref, v_ref):
    y = jnp.dot(x_ref[...], w_ref[...],
                preferred_element_type=jnp.float32)
    y = y.astype(jnp.bfloat16)
    br = y.shape[0]
    hd = _H * _D
    q_part = (y[:, :hd] * jnp.bfloat16(1.0 / np.sqrt(_D))).reshape(br, _H, _D)
    q_ref[...] = q_part.transpose(1, 0, 2)
    k_part = y[:, hd:2 * hd].reshape(br, _H, _D)
    v_part = y[:, 2 * hd:].reshape(br, _H, _D)
    k_ref[...] = k_part.transpose(1, 0, 2)
    ones = jnp.ones((br, _H, 1), jnp.bfloat16)
    zpad = jnp.zeros((br, _H, _D - 1), jnp.bfloat16)
    v_ref[...] = jnp.concatenate([v_part, ones, zpad], axis=-1).transpose(1, 0, 2)


def _attn_kernel(tab_ref, q_ref, k_ref, v_ref, mask_ref, wo_ref, o_ref,
                 acc_ref):
    p = pl.program_id(0)
    ki = tab_ref[p, 1]
    is_first = tab_ref[p, 2] == 1
    is_last = tab_ref[p, 3] == 1

    @pl.when(is_first)
    def _():
        acc_ref[...] = jnp.zeros((_H, _BQ, _D + 1), jnp.float32)

    q = q_ref[...]
    k = k_ref[:, pl.ds(ki * _BK, _BK), :]
    va = v_ref[:, pl.ds(ki * _BK, _BK), :_D + 1]
    s = jax.lax.dot_general(q, k, (((2,), (2,)), ((0,), (0,))),
                            preferred_element_type=jnp.float32)
    pexp = jnp.exp(s + mask_ref[0][None, :, :])
    pv = jax.lax.dot_general(pexp.astype(jnp.bfloat16), va,
                             (((2,), (1,)), ((0,), (0,))),
                             preferred_element_type=jnp.float32)
    acc_ref[...] += pv

    @pl.when(is_last)
    def _():
        o = acc_ref[:, :, :_D] / acc_ref[:, :, _D:_D + 1]
        o2 = jnp.transpose(o, (1, 0, 2)).reshape(_BQ, _H * _D)
        o_ref[...] = jnp.dot(o2.astype(jnp.bfloat16), wo_ref[...],
                             preferred_element_type=jnp.float32)


@jax.jit
def kernel(x, W_q, W_k, W_v, W_o):
    b, s, dim = x.shape
    x2 = x.reshape(s, dim).astype(jnp.bfloat16)
    w_qkv = jnp.concatenate([W_q, W_k, W_v], axis=1).astype(jnp.bfloat16)

    br = 256
    q, k, v = pl.pallas_call(
        _proj_kernel,
        grid=(s // br,),
        in_specs=[
            pl.BlockSpec((br, dim), lambda i: (i, 0)),
            pl.BlockSpec((dim, 3 * _H * _D), lambda i: (0, 0)),
        ],
        out_specs=[
            pl.BlockSpec((_H, br, _D), lambda i: (0, i, 0)),
            pl.BlockSpec((_H, br, _D), lambda i: (0, i, 0)),
            pl.BlockSpec((_H, br, 2 * _D), lambda i: (0, i, 0)),
        ],
        out_shape=[
            jax.ShapeDtypeStruct((_H, s, _D), jnp.bfloat16),
            jax.ShapeDtypeStruct((_H, s, _D), jnp.bfloat16),
            jax.ShapeDtypeStruct((_H, s, 2 * _D), jnp.bfloat16),
        ],
    )(x2, w_qkv)

    tab = jnp.asarray(_TAB_NP)
    mask = jnp.asarray(_MASK_NP)

    grid_spec = pltpu.PrefetchScalarGridSpec(
        num_scalar_prefetch=1,
        grid=(_NUM_PAIRS,),
        in_specs=[
            pl.BlockSpec((_H, _BQ, _D), lambda p, t: (0, t[p, 0], 0)),
            pl.BlockSpec((_H, _S, _D), lambda p, t: (0, 0, 0)),
            pl.BlockSpec((_H, _S, 2 * _D), lambda p, t: (0, 0, 0)),
            pl.BlockSpec((1, _BQ, _BK), lambda p, t: (p, 0, 0)),
            pl.BlockSpec((_H * _D, _DIM), lambda p, t: (0, 0)),
        ],
        out_specs=pl.BlockSpec((_BQ, _DIM), lambda p, t: (t[p, 0], 0)),
        scratch_shapes=[
            pltpu.VMEM((_H, _BQ, _D + 1), jnp.float32),
        ],
    )
    out = pl.pallas_call(
        _attn_kernel,
        grid_spec=grid_spec,
        out_shape=jax.ShapeDtypeStruct((s, _DIM), jnp.float32),
    )(tab, q, k, v, mask, W_o.astype(jnp.bfloat16))
    return out.reshape(b, s, dim)


# speedup vs baseline: 1.0317x; 1.0317x over previous
import functools

import numpy as np
import jax
import jax.numpy as jnp
from jax.experimental import pallas as pl
from jax.experimental.pallas import tpu as pltpu

_S = 4096
_WIN = 16
_H = 16
_D = 64
_DIM = 1024
_BQ = 128
_BK = 128
_NEG = -1e30


def _hilbert_coords(seq_len):
    n = 1
    while n * n < seq_len:
        n *= 2
    t = np.arange(seq_len, dtype=np.int64)
    x = np.zeros(seq_len, dtype=np.int64)
    y = np.zeros(seq_len, dtype=np.int64)
    s = 1
    while s < n:
        rx = 1 & (t // 2)
        ry = 1 & (t ^ rx)
        swap = ry == 0
        flip = swap & (rx == 1)
        xf = np.where(flip, s - 1 - x, x)
        yf = np.where(flip, s - 1 - y, y)
        xn = np.where(swap, yf, xf)
        yn = np.where(swap, xf, yf)
        x = xn + s * rx
        y = yn + s * ry
        t = t // 4
        s *= 2
    return np.stack([x, y], axis=-1).astype(np.float32)


def _neighbor_indices(seq_len, window):
    coords = _hilbert_coords(seq_len)
    diff = coords[:, None, :] - coords[None, :, :]
    dist = np.sqrt((diff ** 2).sum(-1))
    order = np.argsort(dist, axis=-1, kind="stable")
    return order[:, : min(window, seq_len)]


def _build_schedule():
    ni = _neighbor_indices(_S, _WIN)
    qb = np.repeat(np.arange(_S) // _BQ, _WIN)
    kb = (ni // _BK).ravel()
    pairs = sorted(set(zip(qb.tolist(), kb.tolist())))
    P = len(pairs)
    tab = np.zeros((P, 4), dtype=np.int32)
    mask = np.full((P, _BQ, _BK), _NEG, dtype=np.float32)
    for p, (qi, ki) in enumerate(pairs):
        tab[p, 0] = qi
        tab[p, 1] = ki
        tab[p, 2] = int(p == 0 or pairs[p - 1][0] != qi)
        tab[p, 3] = int(p == P - 1 or pairs[p + 1][0] != qi)
        rows = ni[qi * _BQ:(qi + 1) * _BQ]
        r, w = np.nonzero((rows // _BK) == ki)
        mask[p, r, rows[r, w] - ki * _BK] = 0.0
    return tab, mask


_TAB_NP, _MASK_NP = _build_schedule()
_NUM_PAIRS = _TAB_NP.shape[0]


def _proj_kernel(x_ref, w_ref, q_ref, kv_ref):
    y = jnp.dot(x_ref[...], w_ref[...],
                preferred_element_type=jnp.float32)
    y = y.astype(jnp.bfloat16)
    br = y.shape[0]
    hd = _H * _D
    q_part = (y[:, :hd] * jnp.bfloat16(1.0 / np.sqrt(_D))).reshape(br, _H, _D)
    q_ref[...] = q_part.transpose(1, 0, 2)
    k_part = y[:, hd:2 * hd].reshape(br, _H, _D)
    v_part = y[:, 2 * hd:].reshape(br, _H, _D)
    kv_ref[...] = jnp.concatenate([k_part, v_part], axis=-1).transpose(1, 0, 2)


def _attn_kernel(tab_ref, q_ref, kv_ref, mask_ref, wo_ref, o_ref,
                 acc_ref, l_ref):
    p = pl.program_id(0)
    ki = tab_ref[p, 1]
    is_first = tab_ref[p, 2] == 1
    is_last = tab_ref[p, 3] == 1

    @pl.when(is_first)
    def _():
        l_ref[...] = jnp.zeros((_H, _BQ, _BK), jnp.float32)
        acc_ref[...] = jnp.zeros((_H, _BQ, _D), jnp.float32)

    q = q_ref[...]
    kv = kv_ref[:, pl.ds(ki * _BK, _BK), :]
    k = kv[:, :, :_D]
    v = kv[:, :, _D:]
    s = jax.lax.dot_general(q, k, (((2,), (2,)), ((0,), (0,))),
                            preferred_element_type=jnp.float32)
    pexp = jnp.exp(s + mask_ref[0][None, :, :])
    l_ref[...] += jnp.broadcast_to(
        jnp.sum(pexp, axis=-1, keepdims=True), s.shape)
    pv = jax.lax.dot_general(pexp.astype(jnp.bfloat16), v,
                             (((2,), (1,)), ((0,), (0,))),
                             preferred_element_type=jnp.float32)
    acc_ref[...] += pv

    @pl.when(is_last)
    def _():
        o = acc_ref[...] / l_ref[:, :, :_D]
        o2 = jnp.transpose(o, (1, 0, 2)).reshape(_BQ, _H * _D)
        o_ref[...] = jnp.dot(o2.astype(jnp.bfloat16), wo_ref[...],
                             preferred_element_type=jnp.float32)


@jax.jit
def kernel(x, W_q, W_k, W_v, W_o):
    b, s, dim = x.shape
    x2 = x.reshape(s, dim).astype(jnp.bfloat16)
    w_qkv = jnp.concatenate([W_q, W_k, W_v], axis=1).astype(jnp.bfloat16)

    br = 256
    q, kv = pl.pallas_call(
        _proj_kernel,
        grid=(s // br,),
        in_specs=[
            pl.BlockSpec((br, dim), lambda i: (i, 0)),
            pl.BlockSpec((dim, 3 * _H * _D), lambda i: (0, 0)),
        ],
        out_specs=[
            pl.BlockSpec((_H, br, _D), lambda i: (0, i, 0)),
            pl.BlockSpec((_H, br, 2 * _D), lambda i: (0, i, 0)),
        ],
        out_shape=[
            jax.ShapeDtypeStruct((_H, s, _D), jnp.bfloat16),
            jax.ShapeDtypeStruct((_H, s, 2 * _D), jnp.bfloat16),
        ],
    )(x2, w_qkv)

    tab = jnp.asarray(_TAB_NP)
    mask = jnp.asarray(_MASK_NP)

    grid_spec = pltpu.PrefetchScalarGridSpec(
        num_scalar_prefetch=1,
        grid=(_NUM_PAIRS,),
        in_specs=[
            pl.BlockSpec((_H, _BQ, _D), lambda p, t: (0, t[p, 0], 0)),
            pl.BlockSpec((_H, _S, 2 * _D), lambda p, t: (0, 0, 0)),
            pl.BlockSpec((1, _BQ, _BK), lambda p, t: (p, 0, 0)),
            pl.BlockSpec((_H * _D, _DIM), lambda p, t: (0, 0)),
        ],
        out_specs=pl.BlockSpec((_BQ, _DIM), lambda p, t: (t[p, 0], 0)),
        scratch_shapes=[
            pltpu.VMEM((_H, _BQ, _D), jnp.float32),
            pltpu.VMEM((_H, _BQ, _BK), jnp.float32),
        ],
    )
    out = pl.pallas_call(
        _attn_kernel,
        grid_spec=grid_spec,
        out_shape=jax.ShapeDtypeStruct((s, _DIM), jnp.float32),
    )(tab, q, kv, mask, W_o.astype(jnp.bfloat16))
    return out.reshape(b, s, dim)


# speedup vs baseline: 1.0478x; 1.0157x over previous
import functools

import numpy as np
import jax
import jax.numpy as jnp
from jax.experimental import pallas as pl
from jax.experimental.pallas import tpu as pltpu

_S = 4096
_WIN = 16
_H = 16
_D = 64
_DIM = 1024
_BQ = 128
_BK = 128
_NEG = -1e30


def _hilbert_coords(seq_len):
    n = 1
    while n * n < seq_len:
        n *= 2
    t = np.arange(seq_len, dtype=np.int64)
    x = np.zeros(seq_len, dtype=np.int64)
    y = np.zeros(seq_len, dtype=np.int64)
    s = 1
    while s < n:
        rx = 1 & (t // 2)
        ry = 1 & (t ^ rx)
        swap = ry == 0
        flip = swap & (rx == 1)
        xf = np.where(flip, s - 1 - x, x)
        yf = np.where(flip, s - 1 - y, y)
        xn = np.where(swap, yf, xf)
        yn = np.where(swap, xf, yf)
        x = xn + s * rx
        y = yn + s * ry
        t = t // 4
        s *= 2
    return np.stack([x, y], axis=-1).astype(np.float32)


def _neighbor_indices(seq_len, window):
    coords = _hilbert_coords(seq_len)
    diff = coords[:, None, :] - coords[None, :, :]
    dist = np.sqrt((diff ** 2).sum(-1))
    order = np.argsort(dist, axis=-1, kind="stable")
    return order[:, : min(window, seq_len)]


def _build_schedule():
    ni = _neighbor_indices(_S, _WIN)
    qb = np.repeat(np.arange(_S) // _BQ, _WIN)
    kb = (ni // _BK).ravel()
    pairs = sorted(set(zip(qb.tolist(), kb.tolist())))
    P = len(pairs)
    tab = np.zeros((P, 4), dtype=np.int32)
    mask = np.full((P, _BQ, _BK), _NEG, dtype=np.float32)
    for p, (qi, ki) in enumerate(pairs):
        tab[p, 0] = qi
        tab[p, 1] = ki
        tab[p, 2] = int(p == 0 or pairs[p - 1][0] != qi)
        tab[p, 3] = int(p == P - 1 or pairs[p + 1][0] != qi)
        rows = ni[qi * _BQ:(qi + 1) * _BQ]
        r, w = np.nonzero((rows // _BK) == ki)
        mask[p, r, rows[r, w] - ki * _BK] = 0.0
    return tab, mask


_TAB_NP, _MASK_NP = _build_schedule()
_NUM_PAIRS = _TAB_NP.shape[0]


def _proj_kernel(x_ref, w_ref, q_ref, kv_ref):
    y = jnp.dot(x_ref[...], w_ref[...],
                preferred_element_type=jnp.float32)
    y = y.astype(jnp.bfloat16)
    br = y.shape[0]
    hd = _H * _D
    q_part = (y[:, :hd] * jnp.bfloat16(1.0 / np.sqrt(_D))).reshape(br, _H, _D)
    q_ref[...] = q_part.transpose(1, 0, 2)
    k_part = y[:, hd:2 * hd].reshape(br, _H, _D)
    v_part = y[:, 2 * hd:].reshape(br, _H, _D)
    kv_ref[...] = jnp.concatenate([k_part, v_part], axis=-1).transpose(1, 0, 2)


def _attn_kernel(tab_ref, q_ref, kv_ref, mask_ref, wo_ref, o_ref,
                 acc_ref, l_ref):
    p = pl.program_id(0)
    ki = tab_ref[p, 1]
    is_first = tab_ref[p, 2] == 1
    is_last = tab_ref[p, 3] == 1

    @pl.when(is_first)
    def _():
        l_ref[...] = jnp.zeros((_H, _BQ, _BK), jnp.float32)
        acc_ref[...] = jnp.zeros((_H, _BQ, _D), jnp.float32)

    q = q_ref[...]
    kv = kv_ref[:, pl.ds(ki * _BK, _BK), :]
    k = kv[:, :, :_D]
    v = kv[:, :, _D:]
    s = jax.lax.dot_general(q, k, (((2,), (2,)), ((0,), (0,))),
                            preferred_element_type=jnp.float32)
    pexp = jnp.exp(s + mask_ref[0][None, :, :])
    l_ref[...] += jnp.broadcast_to(
        jnp.sum(pexp, axis=-1, keepdims=True), s.shape)
    pv = jax.lax.dot_general(pexp.astype(jnp.bfloat16), v,
                             (((2,), (1,)), ((0,), (0,))),
                             preferred_element_type=jnp.float32)
    acc_ref[...] += pv

    @pl.when(is_last)
    def _():
        o = acc_ref[...] / l_ref[:, :, :_D]
        o2 = jnp.transpose(o, (1, 0, 2)).reshape(_BQ, _H * _D)
        o_ref[...] = jnp.dot(o2.astype(jnp.bfloat16), wo_ref[...],
                             preferred_element_type=jnp.float32)


@jax.jit
def kernel(x, W_q, W_k, W_v, W_o):
    b, s, dim = x.shape
    x2 = x.reshape(s, dim).astype(jnp.bfloat16)
    w_qkv = jnp.concatenate([W_q, W_k, W_v], axis=1).astype(jnp.bfloat16)

    br = 512
    q, kv = pl.pallas_call(
        _proj_kernel,
        grid=(s // br,),
        in_specs=[
            pl.BlockSpec((br, dim), lambda i: (i, 0)),
            pl.BlockSpec((dim, 3 * _H * _D), lambda i: (0, 0)),
        ],
        out_specs=[
            pl.BlockSpec((_H, br, _D), lambda i: (0, i, 0)),
            pl.BlockSpec((_H, br, 2 * _D), lambda i: (0, i, 0)),
        ],
        out_shape=[
            jax.ShapeDtypeStruct((_H, s, _D), jnp.bfloat16),
            jax.ShapeDtypeStruct((_H, s, 2 * _D), jnp.bfloat16),
        ],
    )(x2, w_qkv)

    tab = jnp.asarray(_TAB_NP)
    mask = jnp.asarray(_MASK_NP).astype(jnp.bfloat16)

    grid_spec = pltpu.PrefetchScalarGridSpec(
        num_scalar_prefetch=1,
        grid=(_NUM_PAIRS,),
        in_specs=[
            pl.BlockSpec((_H, _BQ, _D), lambda p, t: (0, t[p, 0], 0)),
            pl.BlockSpec((_H, _S, 2 * _D), lambda p, t: (0, 0, 0)),
            pl.BlockSpec((1, _BQ, _BK), lambda p, t: (p, 0, 0)),
            pl.BlockSpec((_H * _D, _DIM), lambda p, t: (0, 0)),
        ],
        out_specs=pl.BlockSpec((_BQ, _DIM), lambda p, t: (t[p, 0], 0)),
        scratch_shapes=[
            pltpu.VMEM((_H, _BQ, _D), jnp.float32),
            pltpu.VMEM((_H, _BQ, _BK), jnp.float32),
        ],
    )
    out = pl.pallas_call(
        _attn_kernel,
        grid_spec=grid_spec,
        out_shape=jax.ShapeDtypeStruct((s, _DIM), jnp.float32),
    )(tab, q, kv, mask, W_o.astype(jnp.bfloat16))
    return out.reshape(b, s, dim)


# speedup vs baseline: 1.0745x; 1.0255x over previous
import functools

import numpy as np
import jax
import jax.numpy as jnp
from jax.experimental import pallas as pl
from jax.experimental.pallas import tpu as pltpu

_S = 4096
_WIN = 16
_H = 16
_D = 64
_DIM = 1024
_BQ = 128
_BK = 128
_NEG = -1e30


def _hilbert_coords(seq_len):
    n = 1
    while n * n < seq_len:
        n *= 2
    t = np.arange(seq_len, dtype=np.int64)
    x = np.zeros(seq_len, dtype=np.int64)
    y = np.zeros(seq_len, dtype=np.int64)
    s = 1
    while s < n:
        rx = 1 & (t // 2)
        ry = 1 & (t ^ rx)
        swap = ry == 0
        flip = swap & (rx == 1)
        xf = np.where(flip, s - 1 - x, x)
        yf = np.where(flip, s - 1 - y, y)
        xn = np.where(swap, yf, xf)
        yn = np.where(swap, xf, yf)
        x = xn + s * rx
        y = yn + s * ry
        t = t // 4
        s *= 2
    return np.stack([x, y], axis=-1).astype(np.float32)


def _neighbor_indices(seq_len, window):
    coords = _hilbert_coords(seq_len)
    diff = coords[:, None, :] - coords[None, :, :]
    dist = np.sqrt((diff ** 2).sum(-1))
    order = np.argsort(dist, axis=-1, kind="stable")
    return order[:, : min(window, seq_len)]


def _build_schedule():
    ni = _neighbor_indices(_S, _WIN)
    qb = np.repeat(np.arange(_S) // _BQ, _WIN)
    kb = (ni // _BK).ravel()
    pairs = sorted(set(zip(qb.tolist(), kb.tolist())))
    P = len(pairs)
    tab = np.zeros((P, 4), dtype=np.int32)
    mask = np.full((P, _BQ, _BK), _NEG, dtype=np.float32)
    for p, (qi, ki) in enumerate(pairs):
        tab[p, 0] = qi
        tab[p, 1] = ki
        tab[p, 2] = int(p == 0 or pairs[p - 1][0] != qi)
        tab[p, 3] = int(p == P - 1 or pairs[p + 1][0] != qi)
        rows = ni[qi * _BQ:(qi + 1) * _BQ]
        r, w = np.nonzero((rows // _BK) == ki)
        mask[p, r, rows[r, w] - ki * _BK] = 0.0
    return tab, mask


_TAB_NP, _MASK_NP = _build_schedule()
_NUM_PAIRS = _TAB_NP.shape[0]


def _proj_kernel(x_ref, w_ref, q_ref, kv_ref):
    y = jnp.dot(x_ref[...].astype(jnp.bfloat16), w_ref[...],
                preferred_element_type=jnp.float32)
    y = y.astype(jnp.bfloat16)
    br = y.shape[0]
    hd = _H * _D
    q_part = (y[:, :hd] * jnp.bfloat16(1.0 / np.sqrt(_D))).reshape(br, _H, _D)
    q_ref[...] = q_part.transpose(1, 0, 2)
    k_part = y[:, hd:2 * hd].reshape(br, _H, _D)
    v_part = y[:, 2 * hd:].reshape(br, _H, _D)
    kv_ref[...] = jnp.concatenate([k_part, v_part], axis=-1).transpose(1, 0, 2)


def _attn_kernel(tab_ref, q_ref, kv_ref, mask_ref, wo_ref, o_ref,
                 acc_ref, l_ref):
    p = pl.program_id(0)
    ki = tab_ref[p, 1]
    is_first = tab_ref[p, 2] == 1
    is_last = tab_ref[p, 3] == 1

    @pl.when(is_first)
    def _():
        l_ref[...] = jnp.zeros((_H, _BQ, _BK), jnp.float32)
        acc_ref[...] = jnp.zeros((_H, _BQ, _D), jnp.float32)

    q = q_ref[...]
    kv = kv_ref[:, pl.ds(ki * _BK, _BK), :]
    k = kv[:, :, :_D]
    v = kv[:, :, _D:]
    s = jax.lax.dot_general(q, k, (((2,), (2,)), ((0,), (0,))),
                            preferred_element_type=jnp.float32)
    pexp = jnp.exp(s + mask_ref[0][None, :, :])
    l_ref[...] += jnp.broadcast_to(
        jnp.sum(pexp, axis=-1, keepdims=True), s.shape)
    pv = jax.lax.dot_general(pexp.astype(jnp.bfloat16), v,
                             (((2,), (1,)), ((0,), (0,))),
                             preferred_element_type=jnp.float32)
    acc_ref[...] += pv

    @pl.when(is_last)
    def _():
        o = acc_ref[...] / l_ref[:, :, :_D]
        o2 = jnp.transpose(o, (1, 0, 2)).reshape(_BQ, _H * _D)
        o_ref[...] = jnp.dot(o2.astype(jnp.bfloat16), wo_ref[...],
                             preferred_element_type=jnp.float32)


@jax.jit
def kernel(x, W_q, W_k, W_v, W_o):
    b, s, dim = x.shape
    x2 = x.reshape(s, dim)
    w_qkv = jnp.concatenate([W_q, W_k, W_v], axis=1).astype(jnp.bfloat16)

    br = 512
    q, kv = pl.pallas_call(
        _proj_kernel,
        grid=(s // br,),
        in_specs=[
            pl.BlockSpec((br, dim), lambda i: (i, 0)),
            pl.BlockSpec((dim, 3 * _H * _D), lambda i: (0, 0)),
        ],
        out_specs=[
            pl.BlockSpec((_H, br, _D), lambda i: (0, i, 0)),
            pl.BlockSpec((_H, br, 2 * _D), lambda i: (0, i, 0)),
        ],
        out_shape=[
            jax.ShapeDtypeStruct((_H, s, _D), jnp.bfloat16),
            jax.ShapeDtypeStruct((_H, s, 2 * _D), jnp.bfloat16),
        ],
    )(x2, w_qkv)

    tab = jnp.asarray(_TAB_NP)
    mask = jnp.asarray(_MASK_NP).astype(jnp.bfloat16)

    grid_spec = pltpu.PrefetchScalarGridSpec(
        num_scalar_prefetch=1,
        grid=(_NUM_PAIRS,),
        in_specs=[
            pl.BlockSpec((_H, _BQ, _D), lambda p, t: (0, t[p, 0], 0)),
            pl.BlockSpec((_H, _S, 2 * _D), lambda p, t: (0, 0, 0)),
            pl.BlockSpec((1, _BQ, _BK), lambda p, t: (p, 0, 0)),
            pl.BlockSpec((_H * _D, _DIM), lambda p, t: (0, 0)),
        ],
        out_specs=pl.BlockSpec((_BQ, _DIM), lambda p, t: (t[p, 0], 0)),
        scratch_shapes=[
            pltpu.VMEM((_H, _BQ, _D), jnp.float32),
            pltpu.VMEM((_H, _BQ, _BK), jnp.float32),
        ],
    )
    out = pl.pallas_call(
        _attn_kernel,
        grid_spec=grid_spec,
        out_shape=jax.ShapeDtypeStruct((s, _DIM), jnp.float32),
    )(tab, q, kv, mask, W_o.astype(jnp.bfloat16))
    return out.reshape(b, s, dim)
